# baseline (speedup 1.0000x reference)
import functools

import jax
import jax.numpy as jnp
from jax import lax
from jax.experimental import pallas as pl
from jax.experimental.pallas import tpu as pltpu
from jax.experimental.pallas import tpu_sc as plsc

N = 10000
NP = 10240
E = 320000
D = 128
NC, NS, L = 2, 16, 16
RB = 1024
_SROW = [0, 1, 2, 4, 6]
_RROW = [1, 0, 3, 5, 7]


_SEG = 8000
_EPT = E // 4
_NSTG = _EPT // _SEG


def _deg_body(eall, deg_out, hist_v, idx_v, red_v, tmp_v, shared_h, stsem):
    c = lax.axis_index("c")
    s = lax.axis_index("s")
    a = 4 * c + lax.rem(s, 4)
    q = lax.div(s, 4)
    r0 = a * (4 * _NSTG) + q * _NSTG

    ones16 = jnp.full((L,), 1.0, dtype=jnp.float32)
    zeros16 = jnp.zeros((L,), dtype=jnp.float32)

    pltpu.async_copy(eall.at[r0], idx_v.at[0], stsem.at[0])

    @pl.loop(0, NP // L, unroll=8)
    def _zero(i):
        hist_v[pl.ds(i * L, L)] = zeros16

    @pl.loop(0, _NSTG)
    def _stage(st):
        bk = lax.rem(st, 2)
        bkn = lax.rem(st + 1, 2)

        @pl.when(st + 1 <= _NSTG - 1)
        def _prefetch():
            pltpu.async_copy(eall.at[r0 + st + 1], idx_v.at[bkn],
                             stsem.at[bkn])

        pltpu.make_async_copy(eall.at[r0 + st], idx_v.at[bk],
                              stsem.at[bk]).wait()

        @pl.loop(0, _SEG // (L * 10))
        def _chunk(t):
            for u in range(10):
                iv = idx_v[bk, 0, pl.ds(t * (L * 10) + u * L, L)]
                plsc.addupdate_scatter(hist_v, [iv], ones16)

    pltpu.sync_copy(hist_v, shared_h.at[s])
    plsc.subcore_barrier()

    W = NP // 4
    cb = lax.div(s, 4) * W
    m = lax.rem(s, 4)
    pltpu.sync_copy(shared_h.at[m, pl.ds(cb, W)], red_v)
    for r in range(1, 4):
        pltpu.sync_copy(shared_h.at[m + 4 * r, pl.ds(cb, W)], tmp_v)

        @pl.loop(0, W // L, unroll=8)
        def _acc(i):
            red_v[pl.ds(i * L, L)] += tmp_v[pl.ds(i * L, L)]

    pltpu.sync_copy(red_v, deg_out.at[pl.ds(a * NP + cb, W)])


_deg_kernel = functools.partial(
    pl.kernel,
    out_type=jax.ShapeDtypeStruct((8 * NP,), jnp.float32),
    mesh=plsc.VectorSubcoreMesh(core_axis_name="c", subcore_axis_name="s"),
    compiler_params=pltpu.CompilerParams(needs_layout_passes=False),
    scratch_types=[
        pltpu.VMEM((NP,), jnp.float32),
        pltpu.VMEM((2, 1, _SEG), jnp.int32),
        pltpu.VMEM((NP // 4,), jnp.float32),
        pltpu.VMEM((NP // 4,), jnp.float32),
        pltpu.VMEM_SHARED((NS, NP), jnp.float32),
        pltpu.SemaphoreType.DMA((2,)),
    ],
)(_deg_body)


def _mm_body(nodes_ref, Wst_ref, bst_ref, deg_ref, u_ref):
    i = pl.program_id(0)
    x = nodes_ref[...]
    for j in range(5):
        sd = deg_ref[pl.ds(_SROW[j] * NP + i * RB, RB)]
        sc = lax.rsqrt(sd + 1.0)[:, None]
        h = jnp.dot(x, Wst_ref[j], preferred_element_type=jnp.float32)
        u_ref[j] = (h + bst_ref[j][None, :]) * sc


def _mm_kernel(nodes_pad, Wst, bst, deg_flat):
    return pl.pallas_call(
        _mm_body,
        grid=(NP // RB,),
        in_specs=[
            pl.BlockSpec((RB, D), lambda i: (i, 0)),
            pl.BlockSpec((5, D, D), lambda i: (0, 0, 0)),
            pl.BlockSpec((5, D), lambda i: (0, 0)),
            pl.BlockSpec((8 * NP,), lambda i: (0,)),
        ],
        out_specs=pl.BlockSpec((5, RB, D), lambda i: (0, i, 0)),
        out_shape=jax.ShapeDtypeStruct((5, NP, D), jnp.float32),
    )(nodes_pad, Wst, bst, deg_flat)


_K = 50
_NCH = E // (NC * NS) // _K
_NSL = 4
_NIB = 6
_RPT = NP // NS


def _scat_body(u, se, re, gs, gr, ae, af, pe, pf, zrows, acct,
               acc_s, ixs, ixd, rows_v, gsem, ssem, isems, isemd):
    c = lax.axis_index("c")
    s = lax.axis_index("s")
    w = c * NS + s
    pairs = [(se, re), (re, se), (gs, gr), (ae, af), (pe, pf)]

    for j in range(5):
        esrc, edst = pairs[j]
        @pl.when(c == 0)
        def _init_u():
            pltpu.sync_copy(u.at[j, pl.ds(s * _RPT, _RPT)],
                            acc_s.at[pl.ds(s * _RPT, _RPT)])

        @pl.when(c != 0)
        def _init_z():
            pltpu.sync_copy(zrows, acc_s.at[pl.ds(s * _RPT, _RPT)])

        plsc.subcore_barrier()

        for p in range(4):
            pltpu.sync_copy(esrc.at[w, p], ixs.at[p])
            pltpu.sync_copy(edst.at[w, p], ixd.at[p])
        for p in range(2):
            pltpu.async_copy(u.at[j].at[ixs.at[p, 0]], rows_v.at[p],
                             gsem.at[p])

        @pl.loop(0, _NCH)
        def _chunk(i):
            sl = lax.rem(i, _NSL)
            sl2 = lax.rem(i + 2, _NSL)
            bx = lax.rem(i, _NIB)
            bx2 = lax.rem(i + 2, _NIB)
            bx4 = lax.rem(i + 4, _NIB)

            @pl.when(i + 2 <= _NCH - 1)
            def _issue_next_gather():
                @pl.when(i >= 2)
                def _frees():
                    pltpu.make_async_copy(
                        rows_v.at[sl2],
                        acc_s.at[ixd.at[bx2, 0]], ssem.at[sl2]).wait()
                    pltpu.make_async_copy(
                        esrc.at[w, i + 2], ixs.at[bx2], isems.at[bx2]).wait()
                    pltpu.make_async_copy(
                        edst.at[w, i + 2], ixd.at[bx2], isemd.at[bx2]).wait()

                pltpu.async_copy(u.at[j].at[ixs.at[bx2, 0]], rows_v.at[sl2],
                                 gsem.at[sl2])

            pltpu.make_async_copy(u.at[j].at[ixs.at[bx, 0]], rows_v.at[sl],
                                  gsem.at[sl]).wait()
            pltpu.async_copy(rows_v.at[sl], acc_s.at[ixd.at[bx, 0]],
                             ssem.at[sl], add=True)

            @pl.when(i + 4 <= _NCH - 1)
            def _prefetch_idx():
                pltpu.async_copy(esrc.at[w, i + 4], ixs.at[bx4],
                                 isems.at[bx4])
                pltpu.async_copy(edst.at[w, i + 4], ixd.at[bx4],
                                 isemd.at[bx4])

        for tail in (_NCH - 4, _NCH - 3, _NCH - 2, _NCH - 1):
            pltpu.make_async_copy(
                rows_v.at[tail % _NSL],
                acc_s.at[ixd.at[tail % _NIB, 0]],
                ssem.at[tail % _NSL]).wait()

        plsc.subcore_barrier()
        pltpu.sync_copy(acc_s.at[pl.ds(s * _RPT, _RPT)],
                        acct.at[c, j, pl.ds(s * _RPT, _RPT)])


_scat_kernel = functools.partial(
    pl.kernel,
    out_type=jax.ShapeDtypeStruct((NC, 5, NP, D), jnp.float32),
    mesh=plsc.VectorSubcoreMesh(core_axis_name="c", subcore_axis_name="s"),
    compiler_params=pltpu.CompilerParams(needs_layout_passes=False),
    scratch_types=[
        pltpu.VMEM_SHARED((NP, D), jnp.float32),
        pltpu.VMEM((_NIB, 1, _K), jnp.int32),
        pltpu.VMEM((_NIB, 1, _K), jnp.int32),
        pltpu.VMEM((_NSL, _K, D), jnp.float32),
        pltpu.SemaphoreType.DMA((_NSL,)),
        pltpu.SemaphoreType.DMA((_NSL,)),
        pltpu.SemaphoreType.DMA((_NIB,)),
        pltpu.SemaphoreType.DMA((_NIB,)),
    ],
)(_scat_body)


def _fin_body(acc_ref, deg_ref, bns_ref, bnb_ref, Wf3_ref, bf2_ref,
              nodes_ref, out_ref):
    i = pl.program_id(0)
    y = nodes_ref[...] + bf2_ref[0][None, :]
    gamma = lax.rsqrt(jnp.float32(1.0 + 1e-5))
    for j in range(5):
        rd = deg_ref[pl.ds(_RROW[j] * NP + i * RB, RB)]
        t = (acc_ref[0, j] + acc_ref[1, j]) * lax.rsqrt(rd + 1.0)[:, None]
        t = t * (bns_ref[j] * gamma)[None, :] + bnb_ref[j][None, :]
        t = jnp.maximum(t, 0.0)
        y = y + jnp.dot(t, Wf3_ref[j], preferred_element_type=jnp.float32)
    out_ref[...] = y


def _fin_kernel(acc, deg, bns, bnb, Wf3, bf2, nodes_pad):
    return pl.pallas_call(
        _fin_body,
        grid=(NP // RB,),
        in_specs=[
            pl.BlockSpec((NC, 5, RB, D), lambda i: (0, 0, i, 0)),
            pl.BlockSpec((8 * NP,), lambda i: (0,)),
            pl.BlockSpec((5, D), lambda i: (0, 0)),
            pl.BlockSpec((5, D), lambda i: (0, 0)),
            pl.BlockSpec((5, D, D), lambda i: (0, 0, 0)),
            pl.BlockSpec((1, D), lambda i: (0, 0)),
            pl.BlockSpec((RB, D), lambda i: (i, 0)),
        ],
        out_specs=pl.BlockSpec((RB, D), lambda i: (i, 0)),
        out_shape=jax.ShapeDtypeStruct((NP, D), jnp.float32),
    )(acc, deg, bns, bnb, Wf3, bf2, nodes_pad)


def kernel(nodes, senders, receivers, grid_senders, grid_receivers,
           active_senders, active_receivers, passive_senders, passive_receivers,
           W1, b1, W2, b2, W3, b3, W4, b4, W5, b5,
           bn_scale, bn_bias, Wf, bf):
    eall = jnp.concatenate([senders, receivers, grid_senders, grid_receivers,
                            active_senders, active_receivers,
                            passive_senders, passive_receivers]
                           ).reshape(8 * E // _SEG, 1, _SEG)
    deg = _deg_kernel(eall)

    nodes_pad = jnp.pad(nodes, ((0, NP - N), (0, 0)))
    Wst = jnp.stack([W1, W2, W3, W4, W5])
    bst = jnp.stack([b1, b2, b3, b4, b5])
    u = _mm_kernel(nodes_pad, Wst, bst, deg)

    esh = (NC * NS, _NCH, 1, _K)
    zrows = jnp.zeros((_RPT, D), jnp.float32)
    acct = _scat_kernel(u,
                        senders.reshape(esh), receivers.reshape(esh),
                        grid_senders.reshape(esh), grid_receivers.reshape(esh),
                        active_senders.reshape(esh),
                        active_receivers.reshape(esh),
                        passive_senders.reshape(esh),
                        passive_receivers.reshape(esh),
                        zrows)

    bns = bn_scale.reshape(5, D)
    bnb = bn_bias.reshape(5, D)
    Wf3 = Wf.reshape(5, D, D)
    bf2 = bf.reshape(1, D)
    out = _fin_kernel(acct, deg, bns, bnb, Wf3, bf2, nodes_pad)
    return out[:N]


# speedup vs baseline: 1.0802x; 1.0802x over previous
import functools

import jax
import jax.numpy as jnp
from jax import lax
from jax.experimental import pallas as pl
from jax.experimental.pallas import tpu as pltpu
from jax.experimental.pallas import tpu_sc as plsc

N = 10000
NP = 10240
E = 320000
D = 128
NC, NS, L = 2, 16, 16
RB = 1024
_SROW = [0, 1, 2, 4, 6]
_RROW = [1, 0, 3, 5, 7]


_SEG = 8000
_EPT = E // 4
_NSTG = _EPT // _SEG


def _deg_body(eall, deg_out, hist_v, idx_v, red_v, tmp_v, shared_h, stsem):
    c = lax.axis_index("c")
    s = lax.axis_index("s")
    a = 4 * c + lax.rem(s, 4)
    q = lax.div(s, 4)
    r0 = a * (4 * _NSTG) + q * _NSTG

    ones16 = jnp.full((L,), 1.0, dtype=jnp.float32)
    zeros16 = jnp.zeros((L,), dtype=jnp.float32)

    pltpu.async_copy(eall.at[r0], idx_v.at[0], stsem.at[0])

    @pl.loop(0, NP // L, unroll=8)
    def _zero(i):
        hist_v[pl.ds(i * L, L)] = zeros16

    @pl.loop(0, _NSTG)
    def _stage(st):
        bk = lax.rem(st, 2)
        bkn = lax.rem(st + 1, 2)

        @pl.when(st + 1 <= _NSTG - 1)
        def _prefetch():
            pltpu.async_copy(eall.at[r0 + st + 1], idx_v.at[bkn],
                             stsem.at[bkn])

        pltpu.make_async_copy(eall.at[r0 + st], idx_v.at[bk],
                              stsem.at[bk]).wait()

        @pl.loop(0, _SEG // (L * 10))
        def _chunk(t):
            for u in range(10):
                iv = idx_v[bk, 0, pl.ds(t * (L * 10) + u * L, L)]
                plsc.addupdate_scatter(hist_v, [iv], ones16)

    pltpu.sync_copy(hist_v, shared_h.at[s])
    plsc.subcore_barrier()

    W = NP // 4
    cb = lax.div(s, 4) * W
    m = lax.rem(s, 4)
    pltpu.sync_copy(shared_h.at[m, pl.ds(cb, W)], red_v)
    for r in range(1, 4):
        pltpu.sync_copy(shared_h.at[m + 4 * r, pl.ds(cb, W)], tmp_v)

        @pl.loop(0, W // L, unroll=8)
        def _acc(i):
            red_v[pl.ds(i * L, L)] += tmp_v[pl.ds(i * L, L)]

    pltpu.sync_copy(red_v, deg_out.at[pl.ds(a * NP + cb, W)])


_deg_kernel = functools.partial(
    pl.kernel,
    out_type=jax.ShapeDtypeStruct((8 * NP,), jnp.float32),
    mesh=plsc.VectorSubcoreMesh(core_axis_name="c", subcore_axis_name="s"),
    compiler_params=pltpu.CompilerParams(needs_layout_passes=False),
    scratch_types=[
        pltpu.VMEM((NP,), jnp.float32),
        pltpu.VMEM((2, 1, _SEG), jnp.int32),
        pltpu.VMEM((NP // 4,), jnp.float32),
        pltpu.VMEM((NP // 4,), jnp.float32),
        pltpu.VMEM_SHARED((NS, NP), jnp.float32),
        pltpu.SemaphoreType.DMA((2,)),
    ],
)(_deg_body)


def _mm_body(nodes_ref, Wst_ref, bst_ref, deg_ref, u_ref):
    i = pl.program_id(0)
    x = nodes_ref[...]
    for j in range(5):
        sd = deg_ref[pl.ds(_SROW[j] * NP + i * RB, RB)]
        sc = lax.rsqrt(sd + 1.0)[:, None]
        h = jnp.dot(x, Wst_ref[j], preferred_element_type=jnp.float32)
        u_ref[j] = (h + bst_ref[j][None, :]) * sc


def _mm_kernel(nodes_pad, Wst, bst, deg_flat):
    return pl.pallas_call(
        _mm_body,
        grid=(NP // RB,),
        in_specs=[
            pl.BlockSpec((RB, D), lambda i: (i, 0)),
            pl.BlockSpec((5, D, D), lambda i: (0, 0, 0)),
            pl.BlockSpec((5, D), lambda i: (0, 0)),
            pl.BlockSpec((8 * NP,), lambda i: (0,)),
        ],
        out_specs=pl.BlockSpec((5, RB, D), lambda i: (0, i, 0)),
        out_shape=jax.ShapeDtypeStruct((5, NP, D), jnp.float32),
    )(nodes_pad, Wst, bst, deg_flat)


_K = 100
_NCH = E // (NC * NS) // _K
_NSL = 3
_NIB = 5
_RPT = NP // NS


def _scat_body(u, se, re, gs, gr, ae, af, pe, pf, zrows, acct,
               acc_s, ixs, ixd, rows_v, gsem, ssem, isems, isemd):
    c = lax.axis_index("c")
    s = lax.axis_index("s")
    w = c * NS + s
    pairs = [(se, re), (re, se), (gs, gr), (ae, af), (pe, pf)]

    for j in range(5):
        esrc, edst = pairs[j]
        @pl.when(c == 0)
        def _init_u():
            pltpu.sync_copy(u.at[j, pl.ds(s * _RPT, _RPT)],
                            acc_s.at[pl.ds(s * _RPT, _RPT)])

        @pl.when(c != 0)
        def _init_z():
            pltpu.sync_copy(zrows, acc_s.at[pl.ds(s * _RPT, _RPT)])

        plsc.subcore_barrier()

        for p in range(3):
            pltpu.sync_copy(esrc.at[w, p], ixs.at[p])
            pltpu.sync_copy(edst.at[w, p], ixd.at[p])
        pltpu.async_copy(u.at[j].at[ixs.at[0, 0]], rows_v.at[0], gsem.at[0])

        @pl.loop(0, _NCH)
        def _chunk(i):
            sl = lax.rem(i, _NSL)
            sln = lax.rem(i + 1, _NSL)
            bx = lax.rem(i, _NIB)
            bxn = lax.rem(i + 1, _NIB)
            bxp = lax.rem(i + 3, _NIB)

            @pl.when(i + 1 <= _NCH - 1)
            def _issue_next_gather():
                @pl.when(i >= 2)
                def _frees():
                    pltpu.make_async_copy(
                        rows_v.at[sln],
                        acc_s.at[ixd.at[bxn, 0]], ssem.at[sln]).wait()
                    pltpu.make_async_copy(
                        esrc.at[w, i + 1], ixs.at[bxn], isems.at[bxn]).wait()
                    pltpu.make_async_copy(
                        edst.at[w, i + 1], ixd.at[bxn], isemd.at[bxn]).wait()

                pltpu.async_copy(u.at[j].at[ixs.at[bxn, 0]], rows_v.at[sln],
                                 gsem.at[sln])

            pltpu.make_async_copy(u.at[j].at[ixs.at[bx, 0]], rows_v.at[sl],
                                  gsem.at[sl]).wait()
            pltpu.async_copy(rows_v.at[sl], acc_s.at[ixd.at[bx, 0]],
                             ssem.at[sl], add=True)

            @pl.when(i + 3 <= _NCH - 1)
            def _prefetch_idx():
                pltpu.async_copy(esrc.at[w, i + 3], ixs.at[bxp],
                                 isems.at[bxp])
                pltpu.async_copy(edst.at[w, i + 3], ixd.at[bxp],
                                 isemd.at[bxp])

        for tail in (_NCH - 3, _NCH - 2, _NCH - 1):
            pltpu.make_async_copy(
                rows_v.at[tail % _NSL],
                acc_s.at[ixd.at[tail % _NIB, 0]],
                ssem.at[tail % _NSL]).wait()

        plsc.subcore_barrier()
        pltpu.sync_copy(acc_s.at[pl.ds(s * _RPT, _RPT)],
                        acct.at[c, j, pl.ds(s * _RPT, _RPT)])


_scat_kernel = functools.partial(
    pl.kernel,
    out_type=jax.ShapeDtypeStruct((NC, 5, NP, D), jnp.float32),
    mesh=plsc.VectorSubcoreMesh(core_axis_name="c", subcore_axis_name="s"),
    compiler_params=pltpu.CompilerParams(needs_layout_passes=False),
    scratch_types=[
        pltpu.VMEM_SHARED((NP, D), jnp.float32),
        pltpu.VMEM((_NIB, 1, _K), jnp.int32),
        pltpu.VMEM((_NIB, 1, _K), jnp.int32),
        pltpu.VMEM((_NSL, _K, D), jnp.float32),
        pltpu.SemaphoreType.DMA((_NSL,)),
        pltpu.SemaphoreType.DMA((_NSL,)),
        pltpu.SemaphoreType.DMA((_NIB,)),
        pltpu.SemaphoreType.DMA((_NIB,)),
    ],
)(_scat_body)


def _fin_body(acc_ref, deg_ref, bns_ref, bnb_ref, Wf3_ref, bf2_ref,
              nodes_ref, out_ref):
    i = pl.program_id(0)
    y = nodes_ref[...] + bf2_ref[0][None, :]
    gamma = lax.rsqrt(jnp.float32(1.0 + 1e-5))
    for j in range(5):
        rd = deg_ref[pl.ds(_RROW[j] * NP + i * RB, RB)]
        t = (acc_ref[0, j] + acc_ref[1, j]) * lax.rsqrt(rd + 1.0)[:, None]
        t = t * (bns_ref[j] * gamma)[None, :] + bnb_ref[j][None, :]
        t = jnp.maximum(t, 0.0)
        y = y + jnp.dot(t, Wf3_ref[j], preferred_element_type=jnp.float32)
    out_ref[...] = y


def _fin_kernel(acc, deg, bns, bnb, Wf3, bf2, nodes_pad):
    return pl.pallas_call(
        _fin_body,
        grid=(NP // RB,),
        in_specs=[
            pl.BlockSpec((NC, 5, RB, D), lambda i: (0, 0, i, 0)),
            pl.BlockSpec((8 * NP,), lambda i: (0,)),
            pl.BlockSpec((5, D), lambda i: (0, 0)),
            pl.BlockSpec((5, D), lambda i: (0, 0)),
            pl.BlockSpec((5, D, D), lambda i: (0, 0, 0)),
            pl.BlockSpec((1, D), lambda i: (0, 0)),
            pl.BlockSpec((RB, D), lambda i: (i, 0)),
        ],
        out_specs=pl.BlockSpec((RB, D), lambda i: (i, 0)),
        out_shape=jax.ShapeDtypeStruct((NP, D), jnp.float32),
    )(acc, deg, bns, bnb, Wf3, bf2, nodes_pad)


def kernel(nodes, senders, receivers, grid_senders, grid_receivers,
           active_senders, active_receivers, passive_senders, passive_receivers,
           W1, b1, W2, b2, W3, b3, W4, b4, W5, b5,
           bn_scale, bn_bias, Wf, bf):
    eall = jnp.concatenate([senders, receivers, grid_senders, grid_receivers,
                            active_senders, active_receivers,
                            passive_senders, passive_receivers]
                           ).reshape(8 * E // _SEG, 1, _SEG)
    deg = _deg_kernel(eall)

    nodes_pad = jnp.pad(nodes, ((0, NP - N), (0, 0)))
    Wst = jnp.stack([W1, W2, W3, W4, W5])
    bst = jnp.stack([b1, b2, b3, b4, b5])
    u = _mm_kernel(nodes_pad, Wst, bst, deg)

    esh = (NC * NS, _NCH, 1, _K)
    zrows = jnp.zeros((_RPT, D), jnp.float32)
    acct = _scat_kernel(u,
                        senders.reshape(esh), receivers.reshape(esh),
                        grid_senders.reshape(esh), grid_receivers.reshape(esh),
                        active_senders.reshape(esh),
                        active_receivers.reshape(esh),
                        passive_senders.reshape(esh),
                        passive_receivers.reshape(esh),
                        zrows)

    bns = bn_scale.reshape(5, D)
    bnb = bn_bias.reshape(5, D)
    Wf3 = Wf.reshape(5, D, D)
    bf2 = bf.reshape(1, D)
    out = _fin_kernel(acct, deg, bns, bnb, Wf3, bf2, nodes_pad)
    return out[:N]
